# baseline (speedup 1.0000x reference)
import functools
import math

import jax
import jax.numpy as jnp
from jax import lax
from jax.experimental import pallas as pl
from jax.experimental.pallas import tpu as pltpu
from jax.experimental.pallas import tpu_sc as plsc

NUM_CORES = 2
NUM_SUBCORES = 16
NW = NUM_CORES * NUM_SUBCORES
LANES = 16
CHUNK = 64


def _make_kernel(B, S, D, V):
    N = B * S
    n_per_w = N // NW
    n_chunks = n_per_w // CHUNK
    scale = jnp.float32(math.sqrt(float(D)))
    d_regs = D // LANES

    mesh = plsc.VectorSubcoreMesh(
        core_axis_name="c", subcore_axis_name="s",
        num_cores=NUM_CORES, num_subcores=NUM_SUBCORES)

    @functools.partial(
        pl.kernel,
        mesh=mesh,
        out_type=jax.ShapeDtypeStruct((N, D), jnp.float32),
        scratch_types=[
            pltpu.VMEM((n_per_w,), jnp.int32),
            pltpu.VMEM((CHUNK, D), jnp.float32),
            pltpu.VMEM((CHUNK, D), jnp.float32),
            pltpu.SemaphoreType.DMA,
        ],
    )
    def emb_kernel(x_hbm, wt_hbm, pos_hbm, out_hbm, idx_v, wbuf, pbuf, sem):
        wid = lax.axis_index("s") * NUM_CORES + lax.axis_index("c")
        base = wid * n_per_w
        s0 = lax.rem(base, S)
        pltpu.sync_copy(x_hbm.at[pl.ds(base, n_per_w)], idx_v)

        def chunk_body(ci, _):
            off = ci * CHUNK
            pltpu.async_copy(
                wt_hbm.at[idx_v.at[pl.ds(off, CHUNK)]], wbuf, sem).wait()
            pltpu.sync_copy(pos_hbm.at[pl.ds(s0 + off, CHUNK)], pbuf)

            def row_body(r, _):
                for j in range(d_regs):
                    sl = pl.ds(j * LANES, LANES)
                    wbuf[r, sl] = (wbuf[r, sl] + pbuf[r, sl]) * scale
                return 0

            lax.fori_loop(0, CHUNK, row_body, 0)
            pltpu.sync_copy(wbuf, out_hbm.at[pl.ds(base + off, CHUNK)])
            return 0

        lax.fori_loop(0, n_chunks, chunk_body, 0)

    return emb_kernel


def kernel(x, word_table, pos_table):
    B, S = x.shape
    V, D = word_table.shape
    emb = _make_kernel(B, S, D, V)
    out = emb(x.reshape(B * S), word_table, pos_table[:S])
    return out.reshape(B, S, D)


# speedup vs baseline: 1.4838x; 1.4838x over previous
import functools
import math

import jax
import jax.numpy as jnp
from jax import lax
from jax.experimental import pallas as pl
from jax.experimental.pallas import tpu as pltpu
from jax.experimental.pallas import tpu_sc as plsc

NUM_CORES = 2
NUM_SUBCORES = 16
NW = NUM_CORES * NUM_SUBCORES
LANES = 16
CHUNK = 32


def _make_kernel(B, S, D, V):
    s_per_w = S // NW
    n_chunks = s_per_w // CHUNK
    n_units = n_chunks * B
    scale = jnp.float32(math.sqrt(float(D)))
    d_regs = D // LANES

    mesh = plsc.VectorSubcoreMesh(
        core_axis_name="c", subcore_axis_name="s",
        num_cores=NUM_CORES, num_subcores=NUM_SUBCORES)

    @functools.partial(
        pl.kernel,
        mesh=mesh,
        out_type=jax.ShapeDtypeStruct((B * S, D), jnp.float32),
        scratch_types=[
            pltpu.VMEM((B, s_per_w), jnp.int32),
            pltpu.VMEM((CHUNK, D), jnp.float32),
            pltpu.VMEM((CHUNK, D), jnp.float32),
            pltpu.VMEM((CHUNK, D), jnp.float32),
            pltpu.SemaphoreType.DMA,
            pltpu.SemaphoreType.DMA,
        ],
    )
    def emb_kernel(x_hbm, wt_hbm, pos_hbm, out_hbm,
                   idx_v, wbuf0, wbuf1, pbuf, gsem, ssem):
        wid = lax.axis_index("s") * NUM_CORES + lax.axis_index("c")
        s_base = wid * s_per_w
        for b in range(B):
            pltpu.sync_copy(x_hbm.at[b, pl.ds(s_base, s_per_w)],
                            idx_v.at[b])

        wbufs = (wbuf0, wbuf1)

        def start_gather(u, buf):
            bb = lax.rem(u, B)
            ci = u // B
            pltpu.async_copy(
                wt_hbm.at[idx_v.at[bb, pl.ds(ci * CHUNK, CHUNK)]], buf, gsem)

        def wait_gather(buf):
            pltpu.make_async_copy(wt_hbm.at[pl.ds(0, CHUNK)], buf, gsem).wait()

        def start_store(u, buf):
            bb = lax.rem(u, B)
            ci = u // B
            row = bb * S + s_base + ci * CHUNK
            pltpu.async_copy(buf, out_hbm.at[pl.ds(row, CHUNK)], ssem)

        def wait_store(buf):
            pltpu.make_async_copy(buf, out_hbm.at[pl.ds(0, CHUNK)], ssem).wait()

        start_gather(0, wbuf0)

        def pair_body(up, _):
            for p in range(2):
                u = up * 2 + p
                wb = wbufs[p]
                wb_other = wbufs[1 - p]

                @pl.when(u < n_units - 1)
                def _():
                    @pl.when(u >= 1)
                    def _():
                        wait_store(wb_other)
                    start_gather(u + 1, wb_other)

                wait_gather(wb)

                @pl.when(lax.rem(u, B) == 0)
                def _():
                    ci = u // B
                    pltpu.sync_copy(
                        pos_hbm.at[pl.ds(s_base + ci * CHUNK, CHUNK)], pbuf)

                def row_body(r, _):
                    for j in range(d_regs):
                        sl = pl.ds(j * LANES, LANES)
                        wb[r, sl] = (wb[r, sl] + pbuf[r, sl]) * scale
                    return 0

                lax.fori_loop(0, CHUNK, row_body, 0)
                start_store(u, wb)
            return 0

        lax.fori_loop(0, n_units // 2, pair_body, 0)
        wait_store(wbuf0)
        wait_store(wbuf1)

    return emb_kernel


def kernel(x, word_table, pos_table):
    B, S = x.shape
    V, D = word_table.shape
    emb = _make_kernel(B, S, D, V)
    out = emb(x, word_table, pos_table[:S])
    return out.reshape(B, S, D)
